# baseline (speedup 1.0000x reference)
import functools

import jax
import jax.numpy as jnp
from jax import lax
from jax.experimental import pallas as pl
from jax.experimental.pallas import tpu as pltpu
from jax.experimental.pallas import tpu_sc as plsc

B = 4096
HIST = 50
D = 32
NC, NS = 2, 16
NW = NC * NS
BW = B // NW
IB = BW * HIST
L = 16
NG = BW // L
RT = D // 8


def _extract_transpose(lines_v, col_v, blk_v, lanes):
    for g in range(NG):
        rows = lanes + (g * L)
        cols = col_v[pl.ds(g * L, L)]
        for d in range(D):
            vals = plsc.load_gather(lines_v, [rows, cols + d])
            blk_v[d, pl.ds(g * L, L)] = vals


def _body(user_id_hbm, items_hbm, user_t_hbm, item_t_hbm,
          user_out_hbm, item_out_hbm,
          uidx_v, iidx_v, lid_v, col_v, lines_v, blk_v, sem):
    wid = lax.axis_index("s") * NC + lax.axis_index("c")
    b0 = wid * BW

    pltpu.sync_copy(user_id_hbm.at[pl.ds(b0, BW)], uidx_v)
    pltpu.sync_copy(items_hbm.at[pl.ds(b0 * HIST, IB)], iidx_v)

    lanes = lax.iota(jnp.int32, L)
    lanes50 = lanes * HIST

    for g in range(NG):
        raw = uidx_v[pl.ds(g * L, L)]
        lid_v[pl.ds(g * L, L)] = lax.shift_right_logical(raw, 2)
        col_v[pl.ds(g * L, L)] = lax.shift_left(
            lax.bitwise_and(raw, jnp.int32(3)), 5)
    pltpu.async_copy(user_t_hbm.at[lid_v], lines_v, sem).wait()
    _extract_transpose(lines_v, col_v, blk_v, lanes)
    pltpu.sync_copy(blk_v, user_out_hbm.at[:, pl.ds(b0, BW)])

    def plane(l, carry):
        for g in range(NG):
            raw = plsc.load_gather(iidx_v, [lanes50 + (g * L * HIST + l)])
            lid_v[pl.ds(g * L, L)] = lax.shift_right_logical(raw, 2)
            col_v[pl.ds(g * L, L)] = lax.shift_left(
                lax.bitwise_and(raw, jnp.int32(3)), 5)
        pltpu.async_copy(item_t_hbm.at[lid_v], lines_v, sem).wait()
        _extract_transpose(lines_v, col_v, blk_v, lanes)
        pltpu.sync_copy(blk_v, item_out_hbm.at[l, :, pl.ds(b0, BW)])
        return carry

    lax.fori_loop(0, HIST, plane, 0)


_grid_kernel = functools.partial(
    pl.kernel,
    out_type=(
        jax.ShapeDtypeStruct((D, B), jnp.float32),
        jax.ShapeDtypeStruct((HIST, D, B), jnp.float32),
    ),
    mesh=plsc.VectorSubcoreMesh(core_axis_name="c", subcore_axis_name="s",
                                num_cores=NC, num_subcores=NS),
    scratch_types=[
        pltpu.VMEM((BW,), jnp.int32),
        pltpu.VMEM((IB,), jnp.int32),
        pltpu.VMEM((BW,), jnp.int32),
        pltpu.VMEM((BW,), jnp.int32),
        pltpu.VMEM((BW, 128), jnp.float32),
        pltpu.VMEM((D, BW), jnp.float32),
        pltpu.SemaphoreType.DMA,
    ],
    compiler_params=pltpu.CompilerParams(use_tc_tiling_on_sc=True,
                                         needs_layout_passes=False),
)(_body)


def kernel(user_id, items_ids, user_table, item_table):
    items_flat = items_ids.reshape(B * HIST)
    user_t = user_table.reshape(user_table.shape[0] * D // 128, 128)
    item_t = item_table.reshape(item_table.shape[0] * D // 128, 128)
    uT, iT = _grid_kernel(user_id, items_flat, user_t, item_t)
    return uT.T, iT.transpose(2, 0, 1)
